# Initial kernel scaffold; baseline (speedup 1.0000x reference)
import jax
import jax.numpy as jnp
from jax.experimental import pallas as pl


def kernel(points, distance_map):
    raise NotImplementedError("write your pallas kernel here")


# speedup vs baseline: 1.3760x; 1.3760x over previous
import functools

import jax
import jax.numpy as jnp
from jax.experimental import pallas as pl
from jax.experimental.pallas import tpu as pltpu

_ROWS = 256
_COLS = 256
_NEDGE = 64
_CHUNK = 64


def _dice_kernel(edges_ref, dmap_ref, out_ref):
    s = pl.program_id(0)

    e = jnp.clip(edges_ref[0] * 255.0, 0.0, 255.0)
    px = e[0:1, :]
    py = e[1:2, :]
    pjx = e[2:3, :]
    pjy = e[3:4, :]

    jj = jax.lax.broadcasted_iota(jnp.int32, (_COLS, _COLS), 0)
    xx = jax.lax.broadcasted_iota(jnp.int32, (_COLS, _COLS), 1)
    l_incl = (jj <= xx).astype(jnp.float32)
    l_strict = (jj < xx).astype(jnp.float32)

    inter = 0.0
    s_true = 0.0
    s_pred = 0.0
    for c in range(_ROWS // _CHUNK):
        ys = (jax.lax.broadcasted_iota(jnp.int32, (_CHUNK, 1), 0)
              .astype(jnp.float32) + float(c * _CHUNK))
        cond = ((py < ys) & (pjy >= ys)) | ((pjy < ys) & (py >= ys))
        dy = pjy - py
        denom = jnp.where(dy == 0.0, 1.0, dy)
        xc = px + (ys - py) / denom * (pjx - px)
        fc = jnp.floor(jnp.clip(xc, 0.0, 255.0))

        fcm = jnp.where(cond, fc, 400.0)
        jbins = (jax.lax.broadcasted_iota(jnp.int32, (_CHUNK, _NEDGE, _COLS), 2)
                 .astype(jnp.float32))
        fcm3 = jax.lax.broadcast_in_dim(fcm, (_CHUNK, _NEDGE, _COLS), (0, 1))
        hist = jnp.sum((fcm3 == jbins).astype(jnp.float32), axis=1)

        a = jax.lax.dot(hist, l_incl, preferred_element_type=jnp.float32)
        b = jax.lax.dot(hist, l_strict, preferred_element_type=jnp.float32)
        r_tot = a[:, _COLS - 1:_COLS]
        m_lim = r_tot - 1.0 - (r_tot - 2.0 * jnp.floor(r_tot * 0.5))
        b_odd = b - 2.0 * jnp.floor(b * 0.5)
        f_odd = (b <= m_lim).astype(jnp.float32)
        f_even = ((a > b).astype(jnp.float32)
                  * ((b + 1.0) <= m_lim).astype(jnp.float32))
        filled = b_odd * f_odd + (1.0 - b_odd) * f_even

        dchunk = dmap_ref[0, c * _CHUNK:(c + 1) * _CHUNK, :]
        binary = (dchunk * 255.0 <= 127.0).astype(jnp.float32)

        inter = inter + jnp.sum(filled * binary)
        s_true = s_true + jnp.sum(filled)
        s_pred = s_pred + jnp.sum(binary)

    smooth = 1e-06
    dice = (2.0 * inter + smooth) / (s_true + s_pred + smooth)
    loss = 1.0 - dice

    @pl.when(s == 0)
    def _init():
        out_ref[...] = jnp.zeros_like(out_ref)

    out_ref[...] += loss * (1.0 / 64.0)


@jax.jit
def _run(edges, dmap):
    nsam = edges.shape[0]
    out = pl.pallas_call(
        _dice_kernel,
        grid=(nsam,),
        in_specs=[
            pl.BlockSpec((1, 8, _NEDGE), lambda s: (s, 0, 0)),
            pl.BlockSpec((1, _ROWS, _COLS), lambda s: (s, 0, 0)),
        ],
        out_specs=pl.BlockSpec((8, 128), lambda s: (0, 0)),
        out_shape=jax.ShapeDtypeStruct((8, 128), jnp.float32),
        compiler_params=pltpu.CompilerParams(
            dimension_semantics=("arbitrary",),
        ),
    )(edges, dmap)
    return out[0, 0]


def kernel(points, distance_map):
    pts = points[:, :, 0, :]
    px = pts[:, :, 0]
    py = pts[:, :, 1]
    pjx = jnp.roll(px, 1, axis=1)
    pjy = jnp.roll(py, 1, axis=1)
    zeros = jnp.zeros_like(px)
    edges = jnp.stack([px, py, pjx, pjy, zeros, zeros, zeros, zeros],
                      axis=1)
    dmap = distance_map[:, :, :, 0]
    return _run(edges, dmap)
